# Initial kernel scaffold; baseline (speedup 1.0000x reference)
import jax
import jax.numpy as jnp
from jax.experimental import pallas as pl


def kernel(efeat, nfeat, edge_index, W1, b1, W2, b2, gamma, beta):
    raise NotImplementedError("write your pallas kernel here")


# speedup vs baseline: 2.3813x; 2.3813x over previous
import functools

import jax
import jax.numpy as jnp
from jax import lax
from jax.experimental import pallas as pl
from jax.experimental.pallas import tpu as pltpu
from jax.experimental.pallas import tpu_sc as plsc

NW = 32
CHUNK = 80


def _proj_body(nf_ref, ws_ref, wd_ref, ps_ref, pd_ref):
    nf = nf_ref[...]
    ps_ref[...] = jnp.dot(nf, ws_ref[...], preferred_element_type=jnp.float32)
    pd_ref[...] = jnp.dot(nf, wd_ref[...], preferred_element_type=jnp.float32)


def _project(nfeat, w1s, w1d):
    n, _ = nfeat.shape
    hid = w1s.shape[1]
    out = jax.ShapeDtypeStruct((n, hid), jnp.float32)
    return pl.pallas_call(_proj_body, out_shape=(out, out))(nfeat, w1s, w1d)


def _make_gather(n, hid, e, n_chunks):
    e_per_w = e // NW
    mesh = plsc.VectorSubcoreMesh(core_axis_name="c", subcore_axis_name="s")

    @functools.partial(
        pl.kernel,
        mesh=mesh,
        compiler_params=pltpu.CompilerParams(use_tc_tiling_on_sc=False),
        out_type=(
            jax.ShapeDtypeStruct((e, hid), jnp.float32),
            jax.ShapeDtypeStruct((e, hid), jnp.float32),
        ),
        scratch_types=[
            pltpu.VMEM((n_chunks, CHUNK), jnp.int32),
            pltpu.VMEM((n_chunks, CHUNK), jnp.int32),
            pltpu.VMEM((CHUNK, hid), jnp.float32),
            pltpu.VMEM((CHUNK, hid), jnp.float32),
            pltpu.SemaphoreType.DMA,
            pltpu.SemaphoreType.DMA,
        ],
    )
    def gather(ps_hbm, pd_hbm, src_hbm, dst_hbm, g_hbm, h_hbm,
               sidx, didx, bufa, bufb, sema, semb):
        wid = lax.axis_index("s") * 2 + lax.axis_index("c")
        base = wid * e_per_w
        pltpu.sync_copy(src_hbm.at[wid], sidx)
        pltpu.sync_copy(dst_hbm.at[wid], didx)

        def step(j, carry):
            cpa = pltpu.async_copy(ps_hbm.at[sidx.at[j]], bufa, sema)
            cpb = pltpu.async_copy(pd_hbm.at[didx.at[j]], bufb, semb)
            cpa.wait()
            cpb.wait()
            pltpu.sync_copy(bufa, g_hbm.at[pl.ds(base + j * CHUNK, CHUNK)])
            pltpu.sync_copy(bufb, h_hbm.at[pl.ds(base + j * CHUNK, CHUNK)])
            return carry

        lax.fori_loop(0, n_chunks, step, 0)

    return gather


def _mlp_body(g_ref, h_ref, ef_ref, w1e_ref, b1_ref, w2_ref, b2_ref,
              gam_ref, bet_ref, o_ref):
    z = jnp.dot(ef_ref[...], w1e_ref[...], preferred_element_type=jnp.float32)
    z = z + g_ref[...] + h_ref[...] + b1_ref[...]
    hact = z * jax.nn.sigmoid(z)
    o = jnp.dot(hact, w2_ref[...], preferred_element_type=jnp.float32)
    o = o + b2_ref[...]
    mu = jnp.mean(o, axis=-1, keepdims=True)
    c = o - mu
    var = jnp.mean(c * c, axis=-1, keepdims=True)
    o_ref[...] = c * lax.rsqrt(var + 1e-5) * gam_ref[...] + bet_ref[...]


def _mlp(g, h, efeat, w1e, b1, w2, b2, gamma, beta, blk):
    e, hid = g.shape
    efd = efeat.shape[1]
    out_d = w2.shape[1]
    grid = e // blk
    return pl.pallas_call(
        _mlp_body,
        grid=(grid,),
        in_specs=[
            pl.BlockSpec((blk, hid), lambda i: (i, 0)),
            pl.BlockSpec((blk, hid), lambda i: (i, 0)),
            pl.BlockSpec((blk, efd), lambda i: (i, 0)),
            pl.BlockSpec((efd, hid), lambda i: (0, 0)),
            pl.BlockSpec((1, hid), lambda i: (0, 0)),
            pl.BlockSpec((hid, out_d), lambda i: (0, 0)),
            pl.BlockSpec((1, out_d), lambda i: (0, 0)),
            pl.BlockSpec((1, out_d), lambda i: (0, 0)),
            pl.BlockSpec((1, out_d), lambda i: (0, 0)),
        ],
        out_specs=pl.BlockSpec((blk, out_d), lambda i: (i, 0)),
        out_shape=jax.ShapeDtypeStruct((e, out_d), jnp.float32),
    )(g, h, efeat, w1e, b1, w2, b2, gamma, beta)


def kernel(efeat, nfeat, edge_index, W1, b1, W2, b2, gamma, beta):
    e, efd = efeat.shape
    n, nfd = nfeat.shape
    hid = W1.shape[1]

    w1e = W1[:efd]
    w1s = W1[efd:efd + nfd]
    w1d = W1[efd + nfd:]

    ps, pd = _project(nfeat, w1s, w1d)

    e_per_w = e // NW
    n_chunks = e_per_w // CHUNK
    src = edge_index[0].astype(jnp.int32).reshape(NW, n_chunks, CHUNK)
    dst = edge_index[1].astype(jnp.int32).reshape(NW, n_chunks, CHUNK)

    g, h = _make_gather(n, hid, e, n_chunks)(ps, pd, src, dst)

    return _mlp(g, h, efeat, w1e, b1.reshape(1, -1), W2, b2.reshape(1, -1),
                gamma.reshape(1, -1), beta.reshape(1, -1), blk=4000)


# speedup vs baseline: 2.5162x; 1.0566x over previous
import functools

import jax
import jax.numpy as jnp
from jax import lax
from jax.experimental import pallas as pl
from jax.experimental.pallas import tpu as pltpu
from jax.experimental.pallas import tpu_sc as plsc

NW = 32
CHUNK = 80


def _proj_body(nf_ref, ws_ref, wd_ref, ps_ref, pd_ref):
    nf = nf_ref[...]
    ps_ref[...] = jnp.dot(nf, ws_ref[...], preferred_element_type=jnp.float32)
    pd_ref[...] = jnp.dot(nf, wd_ref[...], preferred_element_type=jnp.float32)


def _project(nfeat, w1s, w1d):
    n, _ = nfeat.shape
    hid = w1s.shape[1]
    out = jax.ShapeDtypeStruct((n, hid), jnp.float32)
    return pl.pallas_call(_proj_body, out_shape=(out, out))(nfeat, w1s, w1d)


K_CH = 5
GROUP = K_CH * CHUNK


def _make_gather(n, hid, e, n_chunks):
    e_per_w = e // NW
    n_groups = n_chunks // K_CH
    mesh = plsc.VectorSubcoreMesh(core_axis_name="c", subcore_axis_name="s")

    @functools.partial(
        pl.kernel,
        mesh=mesh,
        compiler_params=pltpu.CompilerParams(use_tc_tiling_on_sc=False),
        out_type=(
            jax.ShapeDtypeStruct((e, hid), jnp.float32),
            jax.ShapeDtypeStruct((e, hid), jnp.float32),
        ),
        scratch_types=[
            pltpu.VMEM((n_chunks, CHUNK), jnp.int32),
            pltpu.VMEM((n_chunks, CHUNK), jnp.int32),
            pltpu.VMEM((GROUP, hid), jnp.float32),
            pltpu.VMEM((GROUP, hid), jnp.float32),
            pltpu.VMEM((GROUP, hid), jnp.float32),
            pltpu.VMEM((GROUP, hid), jnp.float32),
            pltpu.SemaphoreType.DMA,
            pltpu.SemaphoreType.DMA,
            pltpu.SemaphoreType.DMA,
            pltpu.SemaphoreType.DMA,
        ],
    )
    def gather(ps_hbm, pd_hbm, src_hbm, dst_hbm, g_hbm, h_hbm,
               sidx, didx, a0, b0, a1, b1, sa0, sb0, sa1, sb1):
        wid = lax.axis_index("s") * 2 + lax.axis_index("c")
        base = wid * e_per_w
        pltpu.sync_copy(src_hbm.at[wid], sidx)
        pltpu.sync_copy(dst_hbm.at[wid], didx)

        def fire(grp, abuf, bbuf, sa, sb):
            for k in range(K_CH):
                c = grp * K_CH + k
                sl = pl.ds(k * CHUNK, CHUNK)
                pltpu.async_copy(ps_hbm.at[sidx.at[c]], abuf.at[sl], sa)
                pltpu.async_copy(pd_hbm.at[didx.at[c]], bbuf.at[sl], sb)

        def drain_write(grp, abuf, bbuf, sa, sb):
            pltpu.make_async_copy(g_hbm.at[pl.ds(0, GROUP)], abuf, sa).wait()
            pltpu.make_async_copy(h_hbm.at[pl.ds(0, GROUP)], bbuf, sb).wait()
            row = base + grp * GROUP
            pltpu.sync_copy(abuf, g_hbm.at[pl.ds(row, GROUP)])
            pltpu.sync_copy(bbuf, h_hbm.at[pl.ds(row, GROUP)])

        fire(0, a0, b0, sa0, sb0)

        def body(i, carry):
            g0 = 2 * i
            g1 = g0 + 1
            g2 = g0 + 2

            @pl.when(g1 < n_groups)
            def _():
                fire(g1, a1, b1, sa1, sb1)

            drain_write(g0, a0, b0, sa0, sb0)

            @pl.when(g2 < n_groups)
            def _():
                fire(g2, a0, b0, sa0, sb0)

            @pl.when(g1 < n_groups)
            def _():
                drain_write(g1, a1, b1, sa1, sb1)

            return carry

        lax.fori_loop(0, (n_groups + 1) // 2, body, 0)

    return gather


def _mlp_body(g_ref, h_ref, ef_ref, w1e_ref, b1_ref, w2_ref, b2_ref,
              gam_ref, bet_ref, o_ref):
    z = jnp.dot(ef_ref[...], w1e_ref[...], preferred_element_type=jnp.float32)
    z = z + g_ref[...] + h_ref[...] + b1_ref[...]
    hact = z * jax.nn.sigmoid(z)
    o = jnp.dot(hact, w2_ref[...], preferred_element_type=jnp.float32)
    o = o + b2_ref[...]
    mu = jnp.mean(o, axis=-1, keepdims=True)
    c = o - mu
    var = jnp.mean(c * c, axis=-1, keepdims=True)
    o_ref[...] = c * lax.rsqrt(var + 1e-5) * gam_ref[...] + bet_ref[...]


def _mlp(g, h, efeat, w1e, b1, w2, b2, gamma, beta, blk):
    e, hid = g.shape
    efd = efeat.shape[1]
    out_d = w2.shape[1]
    grid = e // blk
    return pl.pallas_call(
        _mlp_body,
        grid=(grid,),
        in_specs=[
            pl.BlockSpec((blk, hid), lambda i: (i, 0)),
            pl.BlockSpec((blk, hid), lambda i: (i, 0)),
            pl.BlockSpec((blk, efd), lambda i: (i, 0)),
            pl.BlockSpec((efd, hid), lambda i: (0, 0)),
            pl.BlockSpec((1, hid), lambda i: (0, 0)),
            pl.BlockSpec((hid, out_d), lambda i: (0, 0)),
            pl.BlockSpec((1, out_d), lambda i: (0, 0)),
            pl.BlockSpec((1, out_d), lambda i: (0, 0)),
            pl.BlockSpec((1, out_d), lambda i: (0, 0)),
        ],
        out_specs=pl.BlockSpec((blk, out_d), lambda i: (i, 0)),
        out_shape=jax.ShapeDtypeStruct((e, out_d), jnp.float32),
    )(g, h, efeat, w1e, b1, w2, b2, gamma, beta)


def kernel(efeat, nfeat, edge_index, W1, b1, W2, b2, gamma, beta):
    e, efd = efeat.shape
    n, nfd = nfeat.shape
    hid = W1.shape[1]

    w1e = W1[:efd]
    w1s = W1[efd:efd + nfd]
    w1d = W1[efd + nfd:]

    ps, pd = _project(nfeat, w1s, w1d)

    e_per_w = e // NW
    n_chunks = e_per_w // CHUNK
    src = edge_index[0].astype(jnp.int32).reshape(NW, n_chunks, CHUNK)
    dst = edge_index[1].astype(jnp.int32).reshape(NW, n_chunks, CHUNK)

    g, h = _make_gather(n, hid, e, n_chunks)(ps, pd, src, dst)

    return _mlp(g, h, efeat, w1e, b1.reshape(1, -1), W2, b2.reshape(1, -1),
                gamma.reshape(1, -1), beta.reshape(1, -1), blk=4000)


# speedup vs baseline: 3.1616x; 1.2565x over previous
import functools

import jax
import jax.numpy as jnp
from jax import lax
from jax.experimental import pallas as pl
from jax.experimental.pallas import tpu as pltpu
from jax.experimental.pallas import tpu_sc as plsc

NW = 32
CHUNK = 80
K_CH = 5
GROUP = K_CH * CHUNK


def _proj_body(nf_ref, ws_ref, wd_ref, ps_ref, pd_ref):
    nf = nf_ref[...]
    ps_ref[...] = jnp.dot(nf, ws_ref[...], preferred_element_type=jnp.float32)
    pd_ref[...] = jnp.dot(nf, wd_ref[...], preferred_element_type=jnp.float32)


def _project(nfeat, w1s, w1d):
    n, _ = nfeat.shape
    hid = w1s.shape[1]
    out = jax.ShapeDtypeStruct((n, hid), jnp.float32)
    return pl.pallas_call(_proj_body, out_shape=(out, out))(nfeat, w1s, w1d)


def _make_gather(n, hid, e, n_chunks):
    e_per_w = e // NW
    n_groups = n_chunks // K_CH
    wide = 2 * hid
    mesh = plsc.VectorSubcoreMesh(core_axis_name="c", subcore_axis_name="s")

    @functools.partial(
        pl.kernel,
        mesh=mesh,
        compiler_params=pltpu.CompilerParams(use_tc_tiling_on_sc=False),
        out_type=(
            jax.ShapeDtypeStruct((e, hid), jnp.float32),
            jax.ShapeDtypeStruct((e, hid), jnp.float32),
        ),
        scratch_types=[
            pltpu.VMEM((n_chunks, CHUNK), jnp.int32),
            pltpu.VMEM((n_chunks, CHUNK), jnp.int32),
            pltpu.VMEM((GROUP, hid), jnp.float32),
            pltpu.VMEM((GROUP, hid), jnp.float32),
            pltpu.VMEM((GROUP, hid), jnp.float32),
            pltpu.VMEM((GROUP, hid), jnp.float32),
            pltpu.SemaphoreType.DMA,
            pltpu.SemaphoreType.DMA,
            pltpu.SemaphoreType.DMA,
            pltpu.SemaphoreType.DMA,
        ],
    )
    def gather(ps_hbm, pd_hbm, src_hbm, dst_hbm, g_hbm, h_hbm,
               sidx, didx, a0, b0, a1, b1, sa0, sb0, sa1, sb1):
        wid = lax.axis_index("s") * 2 + lax.axis_index("c")
        base = wid * e_per_w
        pltpu.sync_copy(src_hbm.at[wid], sidx)
        pltpu.sync_copy(dst_hbm.at[wid], didx)

        def fire(grp, abuf, bbuf, sa, sb):
            for k in range(K_CH):
                c = grp * K_CH + k
                sl = pl.ds(k * CHUNK, CHUNK)
                pltpu.async_copy(ps_hbm.at[sidx.at[c]], abuf.at[sl], sa)
                pltpu.async_copy(pd_hbm.at[didx.at[c]], bbuf.at[sl], sb)

        def drain_write(grp, abuf, bbuf, sa, sb):
            pltpu.make_async_copy(g_hbm.at[pl.ds(0, GROUP)], abuf, sa).wait()
            pltpu.make_async_copy(h_hbm.at[pl.ds(0, GROUP)], bbuf, sb).wait()
            row = base + grp * GROUP
            pltpu.sync_copy(abuf, g_hbm.at[pl.ds(row, GROUP)])
            pltpu.sync_copy(bbuf, h_hbm.at[pl.ds(row, GROUP)])

        fire(0, a0, b0, sa0, sb0)

        def body(i, carry):
            g0 = 2 * i
            g1 = g0 + 1
            g2 = g0 + 2

            @pl.when(g1 < n_groups)
            def _():
                fire(g1, a1, b1, sa1, sb1)

            drain_write(g0, a0, b0, sa0, sb0)

            @pl.when(g2 < n_groups)
            def _():
                fire(g2, a0, b0, sa0, sb0)

            @pl.when(g1 < n_groups)
            def _():
                drain_write(g1, a1, b1, sa1, sb1)

            return carry

        lax.fori_loop(0, (n_groups + 1) // 2, body, 0)

    return gather


def _mlp_body(g_ref, h_ref, ef_ref, w1_ref, b1_ref, w2_ref, b2_ref,
              gam_ref, bet_ref, avg_ref, o_ref):
    z = jnp.dot(ef_ref[...], w1_ref[...], preferred_element_type=jnp.float32)
    z = z + g_ref[...] + h_ref[...] + b1_ref[...]
    hact = z * jax.nn.sigmoid(z)
    o = jnp.dot(hact, w2_ref[...], preferred_element_type=jnp.float32)
    o = o + b2_ref[...]
    avg = avg_ref[...]
    mu = jnp.dot(o, avg, preferred_element_type=jnp.float32)
    c = o - mu
    var = jnp.dot(c * c, avg, preferred_element_type=jnp.float32)
    y = c * lax.rsqrt(var + 1e-5) * gam_ref[...] + bet_ref[...]
    out_d = y.shape[-1] // 2
    o_ref[0] = y[:, :out_d]
    o_ref[1] = y[:, out_d:]


def _mlp(g2, h2, ef2, w1e, b1, w2, b2, gamma, beta, blk):
    e2, wide = g2.shape
    efd2 = ef2.shape[1]
    out_d = w2.shape[1]
    out2 = 2 * out_d
    grid = e2 // blk

    w1_2 = jax.scipy.linalg.block_diag(w1e, w1e)
    w2_2 = jax.scipy.linalg.block_diag(w2, w2)
    b1_2 = jnp.tile(b1, 2).reshape(1, wide)
    b2_2 = jnp.tile(b2, 2).reshape(1, out2)
    gam2 = jnp.tile(gamma, 2).reshape(1, out2)
    bet2 = jnp.tile(beta, 2).reshape(1, out2)
    blk16 = jnp.full((out_d, out_d), 1.0 / out_d, dtype=jnp.float32)
    avg2 = jax.scipy.linalg.block_diag(blk16, blk16)

    y3 = pl.pallas_call(
        _mlp_body,
        grid=(grid,),
        in_specs=[
            pl.BlockSpec((blk, wide), lambda i: (i, 0)),
            pl.BlockSpec((blk, wide), lambda i: (i, 0)),
            pl.BlockSpec((blk, efd2), lambda i: (i, 0)),
            pl.BlockSpec((efd2, wide), lambda i: (0, 0)),
            pl.BlockSpec((1, wide), lambda i: (0, 0)),
            pl.BlockSpec((wide, out2), lambda i: (0, 0)),
            pl.BlockSpec((1, out2), lambda i: (0, 0)),
            pl.BlockSpec((1, out2), lambda i: (0, 0)),
            pl.BlockSpec((1, out2), lambda i: (0, 0)),
            pl.BlockSpec((out2, out2), lambda i: (0, 0)),
        ],
        out_specs=pl.BlockSpec((2, blk, out_d), lambda i: (0, i, 0)),
        out_shape=jax.ShapeDtypeStruct((2, e2, out_d), jnp.float32),
    )(g2, h2, ef2, w1_2, b1_2, w2_2, b2_2, gam2, bet2, avg2)
    return y3.reshape(2 * e2, out_d)


def kernel(efeat, nfeat, edge_index, W1, b1, W2, b2, gamma, beta):
    e, efd = efeat.shape
    n, nfd = nfeat.shape
    hid = W1.shape[1]
    e2 = e // 2

    w1e = W1[:efd]
    w1s = W1[efd:efd + nfd]
    w1d = W1[efd + nfd:]

    ps, pd = _project(nfeat, w1s, w1d)

    e_per_w = e // NW
    n_chunks = e_per_w // CHUNK
    src = edge_index[0].astype(jnp.int32)
    dst = edge_index[1].astype(jnp.int32)
    src_p = jnp.stack([src[:e2], src[e2:]], axis=1).reshape(NW, n_chunks, CHUNK)
    dst_p = jnp.stack([dst[:e2], dst[e2:]], axis=1).reshape(NW, n_chunks, CHUNK)
    ef2 = jnp.concatenate([efeat[:e2], efeat[e2:]], axis=1)

    g, h = _make_gather(n, hid, e, n_chunks)(ps, pd, src_p, dst_p)
    g2 = g.reshape(e2, 2 * hid)
    h2 = h.reshape(e2, 2 * hid)

    return _mlp(g2, h2, ef2, w1e, b1, W2, b2, gamma, beta, blk=2000)


# speedup vs baseline: 3.3809x; 1.0694x over previous
import functools

import jax
import jax.numpy as jnp
from jax import lax
from jax.experimental import pallas as pl
from jax.experimental.pallas import tpu as pltpu
from jax.experimental.pallas import tpu_sc as plsc

NW = 32
CHUNK = 80
K_CH = 5
GROUP = K_CH * CHUNK


def _proj_body(nf_ref, ws_ref, wd_ref, ps_ref, pd_ref):
    nf = nf_ref[...]
    ps_ref[...] = jnp.dot(nf, ws_ref[...], preferred_element_type=jnp.float32)
    pd_ref[...] = jnp.dot(nf, wd_ref[...], preferred_element_type=jnp.float32)


def _project(nfeat, w1s, w1d):
    n, _ = nfeat.shape
    hid = w1s.shape[1]
    out = jax.ShapeDtypeStruct((n, hid), jnp.float32)
    return pl.pallas_call(_proj_body, out_shape=(out, out))(nfeat, w1s, w1d)


def _make_gather(n, hid, e, n_chunks):
    e_per_w = e // NW
    n_groups = n_chunks // K_CH
    wide = 2 * hid
    mesh = plsc.VectorSubcoreMesh(core_axis_name="c", subcore_axis_name="s")

    @functools.partial(
        pl.kernel,
        mesh=mesh,
        compiler_params=pltpu.CompilerParams(use_tc_tiling_on_sc=False),
        out_type=(
            jax.ShapeDtypeStruct((e, hid), jnp.float32),
            jax.ShapeDtypeStruct((e, hid), jnp.float32),
        ),
        scratch_types=[
            pltpu.VMEM((n_chunks, CHUNK), jnp.int32),
            pltpu.VMEM((n_chunks, CHUNK), jnp.int32),
            pltpu.VMEM((GROUP, hid), jnp.float32),
            pltpu.VMEM((GROUP, hid), jnp.float32),
            pltpu.VMEM((GROUP, hid), jnp.float32),
            pltpu.VMEM((GROUP, hid), jnp.float32),
            pltpu.SemaphoreType.DMA,
            pltpu.SemaphoreType.DMA,
            pltpu.SemaphoreType.DMA,
            pltpu.SemaphoreType.DMA,
        ],
    )
    def gather(ps_hbm, pd_hbm, src_hbm, dst_hbm, g_hbm, h_hbm,
               sidx, didx, a0, b0, a1, b1, sa0, sb0, sa1, sb1):
        wid = lax.axis_index("s") * 2 + lax.axis_index("c")
        base = wid * e_per_w
        pltpu.sync_copy(src_hbm.at[wid], sidx)
        pltpu.sync_copy(dst_hbm.at[wid], didx)

        def fire(grp, abuf, bbuf, sa, sb):
            for k in range(K_CH):
                c = grp * K_CH + k
                sl = pl.ds(k * CHUNK, CHUNK)
                pltpu.async_copy(ps_hbm.at[sidx.at[c]], abuf.at[sl], sa)
                pltpu.async_copy(pd_hbm.at[didx.at[c]], bbuf.at[sl], sb)

        def drain_write(grp, abuf, bbuf, sa, sb):
            pltpu.make_async_copy(g_hbm.at[pl.ds(0, GROUP)], abuf, sa).wait()
            pltpu.make_async_copy(h_hbm.at[pl.ds(0, GROUP)], bbuf, sb).wait()
            row = base + grp * GROUP
            pltpu.sync_copy(abuf, g_hbm.at[pl.ds(row, GROUP)])
            pltpu.sync_copy(bbuf, h_hbm.at[pl.ds(row, GROUP)])

        fire(0, a0, b0, sa0, sb0)

        def body(i, carry):
            g0 = 2 * i
            g1 = g0 + 1
            g2 = g0 + 2

            @pl.when(g1 < n_groups)
            def _():
                fire(g1, a1, b1, sa1, sb1)

            drain_write(g0, a0, b0, sa0, sb0)

            @pl.when(g2 < n_groups)
            def _():
                fire(g2, a0, b0, sa0, sb0)

            @pl.when(g1 < n_groups)
            def _():
                drain_write(g1, a1, b1, sa1, sb1)

            return carry

        lax.fori_loop(0, (n_groups + 1) // 2, body, 0)

    return gather


def _mlp_body(g_ref, h_ref, ef_ref, w1_ref, b1_ref, w2_ref, b2_ref,
              gam_ref, bet_ref, avg_ref, o_ref):
    z = jnp.dot(ef_ref[...], w1_ref[...], preferred_element_type=jnp.float32)
    z = z + g_ref[...] + h_ref[...] + b1_ref[...]
    hact = z * jax.nn.sigmoid(z)
    o = jnp.dot(hact, w2_ref[...], preferred_element_type=jnp.float32)
    o = o + b2_ref[...]
    avg = avg_ref[...]
    mu = jnp.dot(o, avg, preferred_element_type=jnp.float32)
    c = o - mu
    var = jnp.dot(c * c, avg, preferred_element_type=jnp.float32)
    y = c * lax.rsqrt(var + 1e-5) * gam_ref[...] + bet_ref[...]
    o_ref[...] = y


def _mlp(g2, h2, ef2, w1e, b1, w2, b2, gamma, beta, blk):
    e2, wide = g2.shape
    efd2 = ef2.shape[1]
    out_d = w2.shape[1]
    out2 = 2 * out_d
    grid = e2 // blk

    w1_2 = jax.scipy.linalg.block_diag(w1e, w1e)
    w2_2 = jax.scipy.linalg.block_diag(w2, w2)
    b1_2 = jnp.tile(b1, 2).reshape(1, wide)
    b2_2 = jnp.tile(b2, 2).reshape(1, out2)
    gam2 = jnp.tile(gamma, 2).reshape(1, out2)
    bet2 = jnp.tile(beta, 2).reshape(1, out2)
    blk16 = jnp.full((out_d, out_d), 1.0 / out_d, dtype=jnp.float32)
    avg2 = jax.scipy.linalg.block_diag(blk16, blk16)

    y3 = pl.pallas_call(
        _mlp_body,
        grid=(grid,),
        in_specs=[
            pl.BlockSpec((blk, wide), lambda i: (i, 0)),
            pl.BlockSpec((blk, wide), lambda i: (i, 0)),
            pl.BlockSpec((blk, efd2), lambda i: (i, 0)),
            pl.BlockSpec((efd2, wide), lambda i: (0, 0)),
            pl.BlockSpec((1, wide), lambda i: (0, 0)),
            pl.BlockSpec((wide, out2), lambda i: (0, 0)),
            pl.BlockSpec((1, out2), lambda i: (0, 0)),
            pl.BlockSpec((1, out2), lambda i: (0, 0)),
            pl.BlockSpec((1, out2), lambda i: (0, 0)),
            pl.BlockSpec((out2, out2), lambda i: (0, 0)),
        ],
        out_specs=pl.BlockSpec((blk, out2), lambda i: (i, 0)),
        out_shape=jax.ShapeDtypeStruct((e2, out2), jnp.float32),
    )(g2, h2, ef2, w1_2, b1_2, w2_2, b2_2, gam2, bet2, avg2)
    return y3.reshape(2 * e2, out_d)


def kernel(efeat, nfeat, edge_index, W1, b1, W2, b2, gamma, beta):
    e, efd = efeat.shape
    n, nfd = nfeat.shape
    hid = W1.shape[1]
    e2 = e // 2

    w1e = W1[:efd]
    w1s = W1[efd:efd + nfd]
    w1d = W1[efd + nfd:]

    ps, pd = _project(nfeat, w1s, w1d)

    e_per_w = e // NW
    n_chunks = e_per_w // CHUNK
    src_p = edge_index[0].astype(jnp.int32).reshape(NW, n_chunks, CHUNK)
    dst_p = edge_index[1].astype(jnp.int32).reshape(NW, n_chunks, CHUNK)
    ef2 = efeat.reshape(e2, 2 * efd)

    g, h = _make_gather(n, hid, e, n_chunks)(ps, pd, src_p, dst_p)
    g2 = g.reshape(e2, 2 * hid)
    h2 = h.reshape(e2, 2 * hid)

    return _mlp(g2, h2, ef2, w1e, b1, W2, b2, gamma, beta, blk=2000)


# speedup vs baseline: 4.3216x; 1.2783x over previous
import functools

import jax
import jax.numpy as jnp
from jax import lax
from jax.experimental import pallas as pl
from jax.experimental.pallas import tpu as pltpu
from jax.experimental.pallas import tpu_sc as plsc

NW = 32
CHUNK = 80
K_CH = 5
GROUP = K_CH * CHUNK
LANES = 16


def _proj_body(nf_ref, ws_ref, wd_ref, ps_ref, pd_ref):
    nf = nf_ref[...]
    ps_ref[...] = jnp.dot(nf, ws_ref[...], preferred_element_type=jnp.float32)
    pd_ref[...] = jnp.dot(nf, wd_ref[...], preferred_element_type=jnp.float32)


def _project(nfeat, w1s, w1d):
    n, _ = nfeat.shape
    hid = w1s.shape[1]
    out = jax.ShapeDtypeStruct((n, hid), jnp.float32)
    return pl.pallas_call(_proj_body, out_shape=(out, out))(nfeat, w1s, w1d)


def _pack_bf16(p):
    n, hid = p.shape
    return lax.bitcast_convert_type(
        p.astype(jnp.bfloat16).reshape(n, hid // 2, 2), jnp.int32)


def _make_gather(n, hw, e, n_chunks):
    e_per_w = e // NW
    q_per_w = e_per_w // 4
    n_groups = n_chunks // K_CH
    n_col_v = CHUNK // LANES
    mesh = plsc.VectorSubcoreMesh(core_axis_name="c", subcore_axis_name="s")

    @functools.partial(
        pl.kernel,
        mesh=mesh,
        compiler_params=pltpu.CompilerParams(
            use_tc_tiling_on_sc=False, needs_layout_passes=False),
        out_type=(
            jax.ShapeDtypeStruct((e, hw), jnp.int32),
            jax.ShapeDtypeStruct((e, hw), jnp.int32),
        ),
        scratch_types=[
            pltpu.VMEM((n_chunks, CHUNK), jnp.int32),
            pltpu.VMEM((n_chunks, CHUNK), jnp.int32),
            pltpu.VMEM((4, q_per_w), jnp.int32),
            pltpu.VMEM((GROUP, hw), jnp.int32),
            pltpu.VMEM((GROUP, hw), jnp.int32),
            pltpu.VMEM((GROUP, hw), jnp.int32),
            pltpu.VMEM((GROUP, hw), jnp.int32),
            pltpu.SemaphoreType.DMA,
            pltpu.SemaphoreType.DMA,
            pltpu.SemaphoreType.DMA,
            pltpu.SemaphoreType.DMA,
        ],
    )
    def gather(ps_hbm, pd_hbm, src_hbm, dst_hbm, g_hbm, h_hbm,
               sidx, didx, qbuf, a0, b0, a1, b1, sa0, sb0, sa1, sb1):
        wid = lax.axis_index("s") * 2 + lax.axis_index("c")
        base = wid * e_per_w
        lanes = lax.iota(jnp.int32, LANES)

        kv = lanes & 3
        qv = lanes >> 2
        qp4 = CHUNK // 4

        def interleave(q_hbm, idx):
            for k in range(4):
                pltpu.sync_copy(q_hbm.at[k].at[wid], qbuf.at[k])

            def row(c, carry):
                for v in range(n_col_v):
                    qidx = c * qp4 + (LANES // 4) * v + qv
                    x = plsc.load_gather(qbuf, [kv, qidx])
                    idx[c, pl.ds(LANES * v, LANES)] = x
                return carry

            lax.fori_loop(0, n_chunks, row, 0)

        interleave(src_hbm, sidx)
        interleave(dst_hbm, didx)

        def fire(grp, abuf, bbuf, sa, sb):
            for k in range(K_CH):
                c = grp * K_CH + k
                sl = pl.ds(k * CHUNK, CHUNK)
                pltpu.async_copy(ps_hbm.at[sidx.at[c]], abuf.at[sl], sa)
                pltpu.async_copy(pd_hbm.at[didx.at[c]], bbuf.at[sl], sb)

        def drain_write(grp, abuf, bbuf, sa, sb):
            pltpu.make_async_copy(g_hbm.at[pl.ds(0, GROUP)], abuf, sa).wait()
            pltpu.make_async_copy(h_hbm.at[pl.ds(0, GROUP)], bbuf, sb).wait()
            row = base + grp * GROUP
            pltpu.sync_copy(abuf, g_hbm.at[pl.ds(row, GROUP)])
            pltpu.sync_copy(bbuf, h_hbm.at[pl.ds(row, GROUP)])

        fire(0, a0, b0, sa0, sb0)

        def body(i, carry):
            g0 = 2 * i
            g1 = g0 + 1
            g2 = g0 + 2

            @pl.when(g1 < n_groups)
            def _():
                fire(g1, a1, b1, sa1, sb1)

            drain_write(g0, a0, b0, sa0, sb0)

            @pl.when(g2 < n_groups)
            def _():
                fire(g2, a0, b0, sa0, sb0)

            @pl.when(g1 < n_groups)
            def _():
                drain_write(g1, a1, b1, sa1, sb1)

            return carry

        lax.fori_loop(0, (n_groups + 1) // 2, body, 0)

    return gather


_MASK_HI = -65536


def _mlp_body(g_ref, h_ref, e0_ref, e1_ref, e2_ref, e3_ref,
              w1lo_ref, w1hi_ref, b1lo_ref, b1hi_ref,
              w2lo_ref, w2hi_ref, b2_ref, gam_ref, bet_ref, avg_ref, o_ref):
    gw = g_ref[...]
    hw = h_ref[...]
    glo = lax.bitcast_convert_type(gw << 16, jnp.float32)
    ghi = lax.bitcast_convert_type(gw & _MASK_HI, jnp.float32)
    hlo = lax.bitcast_convert_type(hw << 16, jnp.float32)
    hhi = lax.bitcast_convert_type(hw & _MASK_HI, jnp.float32)

    efc = jnp.concatenate(
        [e0_ref[...], e1_ref[...], e2_ref[...], e3_ref[...]], axis=1)
    zlo = jnp.dot(efc, w1lo_ref[...], preferred_element_type=jnp.float32)
    zhi = jnp.dot(efc, w1hi_ref[...], preferred_element_type=jnp.float32)
    zlo = zlo + glo + hlo + b1lo_ref[...]
    zhi = zhi + ghi + hhi + b1hi_ref[...]
    alo = zlo * jax.nn.sigmoid(zlo)
    ahi = zhi * jax.nn.sigmoid(zhi)
    o = (jnp.dot(alo, w2lo_ref[...], preferred_element_type=jnp.float32)
         + jnp.dot(ahi, w2hi_ref[...], preferred_element_type=jnp.float32)
         + b2_ref[...])
    avg = avg_ref[...]
    mu = jnp.dot(o, avg, preferred_element_type=jnp.float32)
    c = o - mu
    var = jnp.dot(c * c, avg, preferred_element_type=jnp.float32)
    y = c * lax.rsqrt(var + 1e-5) * gam_ref[...] + bet_ref[...]
    out_d = y.shape[-1] // 4
    for k in range(4):
        o_ref[k] = y[:, k * out_d:(k + 1) * out_d]


def _bd4(m):
    return jax.scipy.linalg.block_diag(m, m, m, m)


def _mlp(g4, h4, efeat, w1e, b1, w2, b2, gamma, beta, blk):
    e4, wide = g4.shape
    e, efd = efeat.shape
    hid = w1e.shape[1]
    hh = hid // 2
    out_d = w2.shape[1]
    grid = e4 // blk
    qblk = e // 4 // blk

    w1lo = _bd4(w1e[:, :hh])
    w1hi = _bd4(w1e[:, hh:])
    b1lo = jnp.tile(b1[:hh], 4).reshape(1, 4 * hh)
    b1hi = jnp.tile(b1[hh:], 4).reshape(1, 4 * hh)
    w2lo = _bd4(w2[:hh])
    w2hi = _bd4(w2[hh:])
    b2_4 = jnp.tile(b2, 4).reshape(1, 4 * out_d)
    gam4 = jnp.tile(gamma, 4).reshape(1, 4 * out_d)
    bet4 = jnp.tile(beta, 4).reshape(1, 4 * out_d)
    avg4 = _bd4(jnp.full((out_d, out_d), 1.0 / out_d, dtype=jnp.float32))

    ef_spec = [
        pl.BlockSpec((blk, efd), lambda i, k=k: (i + k * qblk, 0))
        for k in range(4)
    ]
    y4 = pl.pallas_call(
        _mlp_body,
        grid=(grid,),
        in_specs=[
            pl.BlockSpec((blk, wide), lambda i: (i, 0)),
            pl.BlockSpec((blk, wide), lambda i: (i, 0)),
            *ef_spec,
            pl.BlockSpec((4 * efd, 4 * hh), lambda i: (0, 0)),
            pl.BlockSpec((4 * efd, 4 * hh), lambda i: (0, 0)),
            pl.BlockSpec((1, 4 * hh), lambda i: (0, 0)),
            pl.BlockSpec((1, 4 * hh), lambda i: (0, 0)),
            pl.BlockSpec((4 * hh, 4 * out_d), lambda i: (0, 0)),
            pl.BlockSpec((4 * hh, 4 * out_d), lambda i: (0, 0)),
            pl.BlockSpec((1, 4 * out_d), lambda i: (0, 0)),
            pl.BlockSpec((1, 4 * out_d), lambda i: (0, 0)),
            pl.BlockSpec((1, 4 * out_d), lambda i: (0, 0)),
            pl.BlockSpec((4 * out_d, 4 * out_d), lambda i: (0, 0)),
        ],
        out_specs=pl.BlockSpec((4, blk, out_d), lambda i: (0, i, 0)),
        out_shape=jax.ShapeDtypeStruct((4, e4, out_d), jnp.float32),
    )(g4, h4, efeat, efeat, efeat, efeat, w1lo, w1hi, b1lo, b1hi,
      w2lo, w2hi, b2_4, gam4, bet4, avg4)
    return y4.reshape(e, out_d)


def kernel(efeat, nfeat, edge_index, W1, b1, W2, b2, gamma, beta):
    e, efd = efeat.shape
    n, nfd = nfeat.shape
    hid = W1.shape[1]
    hh = hid // 2

    w1e = W1[:efd]
    perm = jnp.stack([jnp.arange(hh), jnp.arange(hh) + hh], axis=1).reshape(hid)
    w1s = W1[efd:efd + nfd][:, perm]
    w1d = W1[efd + nfd:][:, perm]

    ps, pd = _project(nfeat, w1s, w1d)
    ps_p = _pack_bf16(ps)
    pd_p = _pack_bf16(pd)

    e_per_w = e // NW
    n_chunks = e_per_w // CHUNK
    src_q = edge_index[0].astype(jnp.int32).reshape(4, NW, e_per_w // 4)
    dst_q = edge_index[1].astype(jnp.int32).reshape(4, NW, e_per_w // 4)

    g, h = _make_gather(n, hid // 2, e, n_chunks)(ps_p, pd_p, src_q, dst_q)
    g4 = g.reshape(e // 4, 2 * hid)
    h4 = h.reshape(e // 4, 2 * hid)

    return _mlp(g4, h4, efeat, w1e, b1, W2, b2, gamma, beta, blk=1000)


# speedup vs baseline: 4.7376x; 1.0963x over previous
import functools

import jax
import jax.numpy as jnp
from jax import lax
from jax.experimental import pallas as pl
from jax.experimental.pallas import tpu as pltpu
from jax.experimental.pallas import tpu_sc as plsc

NW = 32
CHUNK = 80
K_CH = 5
GROUP = K_CH * CHUNK
LANES = 16


def _rn_bf16_hi(x):
    u = lax.bitcast_convert_type(x, jnp.int32)
    r = u + 0x7FFF + ((u >> 16) & 1)
    return r & _MASK_HI


def _proj_body(nf_ref, wsl_ref, wsh_ref, wdl_ref, wdh_ref, ps_ref, pd_ref):
    nf = nf_ref[...]

    def pack(wl_ref, wh_ref):
        zl = jnp.dot(nf, wl_ref[...], preferred_element_type=jnp.float32)
        zh = jnp.dot(nf, wh_ref[...], preferred_element_type=jnp.float32)
        lo = lax.shift_right_logical(_rn_bf16_hi(zl), 16)
        return _rn_bf16_hi(zh) | lo

    ps_ref[...] = pack(wsl_ref, wsh_ref)
    pd_ref[...] = pack(wdl_ref, wdh_ref)


def _project_packed(nfeat, w1s, w1d):
    n, _ = nfeat.shape
    hh = w1s.shape[1] // 2
    out = jax.ShapeDtypeStruct((n, hh), jnp.int32)
    return pl.pallas_call(_proj_body, out_shape=(out, out))(
        nfeat, w1s[:, :hh], w1s[:, hh:], w1d[:, :hh], w1d[:, hh:])


def _make_gather(n, hw, e, n_chunks):
    e_per_w = e // NW
    q_per_w = e_per_w // 4
    n_groups = n_chunks // K_CH
    n_col_v = CHUNK // LANES
    mesh = plsc.VectorSubcoreMesh(core_axis_name="c", subcore_axis_name="s")

    @functools.partial(
        pl.kernel,
        mesh=mesh,
        compiler_params=pltpu.CompilerParams(
            use_tc_tiling_on_sc=False, needs_layout_passes=False),
        out_type=(
            jax.ShapeDtypeStruct((e, hw), jnp.int32),
            jax.ShapeDtypeStruct((e, hw), jnp.int32),
        ),
        scratch_types=[
            pltpu.VMEM((n_chunks, CHUNK), jnp.int32),
            pltpu.VMEM((n_chunks, CHUNK), jnp.int32),
            pltpu.VMEM((4, q_per_w), jnp.int32),
            pltpu.VMEM((GROUP, hw), jnp.int32),
            pltpu.VMEM((GROUP, hw), jnp.int32),
            pltpu.VMEM((GROUP, hw), jnp.int32),
            pltpu.VMEM((GROUP, hw), jnp.int32),
            pltpu.SemaphoreType.DMA,
            pltpu.SemaphoreType.DMA,
            pltpu.SemaphoreType.DMA,
            pltpu.SemaphoreType.DMA,
        ],
    )
    def gather(ps_hbm, pd_hbm, src_hbm, dst_hbm, g_hbm, h_hbm,
               sidx, didx, qbuf, a0, b0, a1, b1, sa0, sb0, sa1, sb1):
        wid = lax.axis_index("s") * 2 + lax.axis_index("c")
        base = wid * e_per_w
        lanes = lax.iota(jnp.int32, LANES)

        kv = lanes & 3
        qv = lanes >> 2
        qp4 = CHUNK // 4

        def interleave(q_hbm, idx):
            for k in range(4):
                pltpu.sync_copy(q_hbm.at[k].at[wid], qbuf.at[k])

            def row(c, carry):
                for v in range(n_col_v):
                    qidx = c * qp4 + (LANES // 4) * v + qv
                    x = plsc.load_gather(qbuf, [kv, qidx])
                    idx[c, pl.ds(LANES * v, LANES)] = x
                return carry

            lax.fori_loop(0, n_chunks, row, 0)

        interleave(src_hbm, sidx)
        interleave(dst_hbm, didx)

        def fire(grp, abuf, bbuf, sa, sb):
            for k in range(K_CH):
                c = grp * K_CH + k
                sl = pl.ds(k * CHUNK, CHUNK)
                pltpu.async_copy(ps_hbm.at[sidx.at[c]], abuf.at[sl], sa)
                pltpu.async_copy(pd_hbm.at[didx.at[c]], bbuf.at[sl], sb)

        def drain_write(grp, abuf, bbuf, sa, sb):
            pltpu.make_async_copy(g_hbm.at[pl.ds(0, GROUP)], abuf, sa).wait()
            pltpu.make_async_copy(h_hbm.at[pl.ds(0, GROUP)], bbuf, sb).wait()
            row = base + grp * GROUP
            pltpu.sync_copy(abuf, g_hbm.at[pl.ds(row, GROUP)])
            pltpu.sync_copy(bbuf, h_hbm.at[pl.ds(row, GROUP)])

        fire(0, a0, b0, sa0, sb0)

        def body(i, carry):
            g0 = 2 * i
            g1 = g0 + 1
            g2 = g0 + 2

            @pl.when(g1 < n_groups)
            def _():
                fire(g1, a1, b1, sa1, sb1)

            drain_write(g0, a0, b0, sa0, sb0)

            @pl.when(g2 < n_groups)
            def _():
                fire(g2, a0, b0, sa0, sb0)

            @pl.when(g1 < n_groups)
            def _():
                drain_write(g1, a1, b1, sa1, sb1)

            return carry

        lax.fori_loop(0, (n_groups + 1) // 2, body, 0)

    return gather


_MASK_HI = -65536


def _mlp_body(g_ref, h_ref, e0_ref, e1_ref, e2_ref, e3_ref,
              w1lo_ref, w1hi_ref, b1lo_ref, b1hi_ref,
              w2lo_ref, w2hi_ref, b2_ref, gam_ref, bet_ref, avg_ref, o_ref):
    gw = g_ref[...]
    hw = h_ref[...]
    glo = lax.bitcast_convert_type(gw << 16, jnp.float32)
    ghi = lax.bitcast_convert_type(gw & _MASK_HI, jnp.float32)
    hlo = lax.bitcast_convert_type(hw << 16, jnp.float32)
    hhi = lax.bitcast_convert_type(hw & _MASK_HI, jnp.float32)

    efc = jnp.concatenate(
        [e0_ref[...], e1_ref[...], e2_ref[...], e3_ref[...]], axis=1)
    zlo = jnp.dot(efc, w1lo_ref[...], preferred_element_type=jnp.float32)
    zhi = jnp.dot(efc, w1hi_ref[...], preferred_element_type=jnp.float32)
    zlo = zlo + glo + hlo + b1lo_ref[...]
    zhi = zhi + ghi + hhi + b1hi_ref[...]
    alo = zlo * jax.nn.sigmoid(zlo)
    ahi = zhi * jax.nn.sigmoid(zhi)
    o = (jnp.dot(alo, w2lo_ref[...], preferred_element_type=jnp.float32)
         + jnp.dot(ahi, w2hi_ref[...], preferred_element_type=jnp.float32)
         + b2_ref[...])
    avg = avg_ref[...]
    mu = jnp.dot(o, avg, preferred_element_type=jnp.float32)
    c = o - mu
    var = jnp.dot(c * c, avg, preferred_element_type=jnp.float32)
    y = c * lax.rsqrt(var + 1e-5) * gam_ref[...] + bet_ref[...]
    out_d = y.shape[-1] // 4
    for k in range(4):
        o_ref[k] = y[:, k * out_d:(k + 1) * out_d]


def _bd4(m):
    return jax.scipy.linalg.block_diag(m, m, m, m)


def _mlp(g4, h4, efeat, w1e, b1, w2, b2, gamma, beta, blk):
    e4, wide = g4.shape
    e, efd = efeat.shape
    hid = w1e.shape[1]
    hh = hid // 2
    out_d = w2.shape[1]
    grid = e4 // blk
    qblk = e // 4 // blk

    w1lo = _bd4(w1e[:, :hh])
    w1hi = _bd4(w1e[:, hh:])
    b1lo = jnp.tile(b1[:hh], 4).reshape(1, 4 * hh)
    b1hi = jnp.tile(b1[hh:], 4).reshape(1, 4 * hh)
    w2lo = _bd4(w2[:hh])
    w2hi = _bd4(w2[hh:])
    b2_4 = jnp.tile(b2, 4).reshape(1, 4 * out_d)
    gam4 = jnp.tile(gamma, 4).reshape(1, 4 * out_d)
    bet4 = jnp.tile(beta, 4).reshape(1, 4 * out_d)
    avg4 = _bd4(jnp.full((out_d, out_d), 1.0 / out_d, dtype=jnp.float32))

    ef_spec = [
        pl.BlockSpec((blk, efd), lambda i, k=k: (i + k * qblk, 0))
        for k in range(4)
    ]
    y4 = pl.pallas_call(
        _mlp_body,
        grid=(grid,),
        in_specs=[
            pl.BlockSpec((blk, wide), lambda i: (i, 0)),
            pl.BlockSpec((blk, wide), lambda i: (i, 0)),
            *ef_spec,
            pl.BlockSpec((4 * efd, 4 * hh), lambda i: (0, 0)),
            pl.BlockSpec((4 * efd, 4 * hh), lambda i: (0, 0)),
            pl.BlockSpec((1, 4 * hh), lambda i: (0, 0)),
            pl.BlockSpec((1, 4 * hh), lambda i: (0, 0)),
            pl.BlockSpec((4 * hh, 4 * out_d), lambda i: (0, 0)),
            pl.BlockSpec((4 * hh, 4 * out_d), lambda i: (0, 0)),
            pl.BlockSpec((1, 4 * out_d), lambda i: (0, 0)),
            pl.BlockSpec((1, 4 * out_d), lambda i: (0, 0)),
            pl.BlockSpec((1, 4 * out_d), lambda i: (0, 0)),
            pl.BlockSpec((4 * out_d, 4 * out_d), lambda i: (0, 0)),
        ],
        out_specs=pl.BlockSpec((4, blk, out_d), lambda i: (0, i, 0)),
        out_shape=jax.ShapeDtypeStruct((4, e4, out_d), jnp.float32),
    )(g4, h4, efeat, efeat, efeat, efeat, w1lo, w1hi, b1lo, b1hi,
      w2lo, w2hi, b2_4, gam4, bet4, avg4)
    return y4.reshape(e, out_d)


def kernel(efeat, nfeat, edge_index, W1, b1, W2, b2, gamma, beta):
    e, efd = efeat.shape
    n, nfd = nfeat.shape
    hid = W1.shape[1]
    hh = hid // 2

    w1e = W1[:efd]
    w1s = W1[efd:efd + nfd]
    w1d = W1[efd + nfd:]

    ps_p, pd_p = _project_packed(nfeat, w1s, w1d)

    e_per_w = e // NW
    n_chunks = e_per_w // CHUNK
    src_q = edge_index[0].astype(jnp.int32).reshape(4, NW, e_per_w // 4)
    dst_q = edge_index[1].astype(jnp.int32).reshape(4, NW, e_per_w // 4)

    g, h = _make_gather(n, hid // 2, e, n_chunks)(ps_p, pd_p, src_q, dst_q)
    g4 = g.reshape(e // 4, 2 * hid)
    h4 = h.reshape(e // 4, 2 * hid)

    return _mlp(g4, h4, efeat, w1e, b1, W2, b2, gamma, beta, blk=1000)


# speedup vs baseline: 7.0442x; 1.4869x over previous
import functools

import jax
import jax.numpy as jnp
from jax import lax
from jax.experimental import pallas as pl
from jax.experimental.pallas import tpu as pltpu
from jax.experimental.pallas import tpu_sc as plsc

NW = 32
CHUNK = 80
K_CH = 5
GROUP = K_CH * CHUNK
LANES = 16


def _rn_bf16_hi(x):
    u = lax.bitcast_convert_type(x, jnp.int32)
    r = u + 0x7FFF + ((u >> 16) & 1)
    return r & _MASK_HI


def _proj_body(nf_ref, wsl_ref, wsh_ref, wdl_ref, wdh_ref, ps_ref, pd_ref):
    nf = nf_ref[...]

    def pack(wl_ref, wh_ref):
        zl = jnp.dot(nf, wl_ref[...], preferred_element_type=jnp.float32)
        zh = jnp.dot(nf, wh_ref[...], preferred_element_type=jnp.float32)
        lo = lax.shift_right_logical(_rn_bf16_hi(zl), 16)
        return _rn_bf16_hi(zh) | lo

    ps_ref[...] = pack(wsl_ref, wsh_ref)
    pd_ref[...] = pack(wdl_ref, wdh_ref)


def _project_packed(nfeat, w1s, w1d):
    n, _ = nfeat.shape
    hh = w1s.shape[1] // 2
    out = jax.ShapeDtypeStruct((n, hh), jnp.int32)
    return pl.pallas_call(_proj_body, out_shape=(out, out))(
        nfeat, w1s[:, :hh], w1s[:, hh:], w1d[:, :hh], w1d[:, hh:])


def _make_gather(n, hw, e, n_chunks):
    e_per_w = e // NW
    q_per_w = e_per_w // 4
    n_groups = n_chunks // K_CH
    n_col_v = CHUNK // LANES
    mesh = plsc.VectorSubcoreMesh(core_axis_name="c", subcore_axis_name="s")

    @functools.partial(
        pl.kernel,
        mesh=mesh,
        compiler_params=pltpu.CompilerParams(
            use_tc_tiling_on_sc=False, needs_layout_passes=False),
        out_type=(
            jax.ShapeDtypeStruct((e, hw), jnp.int32),
            jax.ShapeDtypeStruct((e, hw), jnp.int32),
        ),
        scratch_types=[
            pltpu.VMEM((n_chunks, CHUNK), jnp.int32),
            pltpu.VMEM((n_chunks, CHUNK), jnp.int32),
            pltpu.VMEM((4, q_per_w), jnp.int32),
            pltpu.VMEM((GROUP, hw), jnp.int32),
            pltpu.VMEM((GROUP, hw), jnp.int32),
            pltpu.VMEM((GROUP, hw), jnp.int32),
            pltpu.VMEM((GROUP, hw), jnp.int32),
            pltpu.SemaphoreType.DMA,
            pltpu.SemaphoreType.DMA,
            pltpu.SemaphoreType.DMA,
            pltpu.SemaphoreType.DMA,
        ],
    )
    def gather(ps_hbm, pd_hbm, src_hbm, dst_hbm, g_hbm, h_hbm,
               sidx, didx, qbuf, a0, b0, a1, b1, sa0, sb0, sa1, sb1):
        wid = lax.axis_index("s") * 2 + lax.axis_index("c")
        base = wid * e_per_w
        lanes = lax.iota(jnp.int32, LANES)

        kv = lanes & 3
        qv = lanes >> 2
        qp4 = CHUNK // 4

        def interleave(q_hbm, idx):
            for k in range(4):
                pltpu.sync_copy(q_hbm.at[k].at[wid], qbuf.at[k])

            def row(c, carry):
                for v in range(n_col_v):
                    qidx = c * qp4 + (LANES // 4) * v + qv
                    x = plsc.load_gather(qbuf, [kv, qidx])
                    idx[c, pl.ds(LANES * v, LANES)] = x
                return carry

            lax.fori_loop(0, n_chunks, row, 0)

        interleave(src_hbm, sidx)
        interleave(dst_hbm, didx)

        def fire(grp, abuf, bbuf, sa, sb):
            for k in range(K_CH):
                c = grp * K_CH + k
                sl = pl.ds(k * CHUNK, CHUNK)
                pltpu.async_copy(ps_hbm.at[sidx.at[c]], abuf.at[sl], sa)
                pltpu.async_copy(pd_hbm.at[didx.at[c]], bbuf.at[sl], sb)

        def drain_write(grp, abuf, bbuf, sa, sb):
            pltpu.make_async_copy(g_hbm.at[pl.ds(0, GROUP)], abuf, sa).wait()
            pltpu.make_async_copy(h_hbm.at[pl.ds(0, GROUP)], bbuf, sb).wait()
            row = base + grp * GROUP
            pltpu.sync_copy(abuf, g_hbm.at[pl.ds(row, GROUP)])
            pltpu.sync_copy(bbuf, h_hbm.at[pl.ds(row, GROUP)])

        fire(0, a0, b0, sa0, sb0)

        def body(i, carry):
            g0 = 2 * i
            g1 = g0 + 1
            g2 = g0 + 2

            @pl.when(g1 < n_groups)
            def _():
                fire(g1, a1, b1, sa1, sb1)

            drain_write(g0, a0, b0, sa0, sb0)

            @pl.when(g2 < n_groups)
            def _():
                fire(g2, a0, b0, sa0, sb0)

            @pl.when(g1 < n_groups)
            def _():
                drain_write(g1, a1, b1, sa1, sb1)

            return carry

        lax.fori_loop(0, (n_groups + 1) // 2, body, 0)

    return gather


_MASK_HI = -65536


def _mlp_body(g_ref, h_ref, e0_ref, e1_ref, e2_ref, e3_ref,
              w1lo_ref, w1hi_ref, b1lo_ref, b1hi_ref,
              w2lo_ref, w2hi_ref, b2_ref, gam_ref, bet_ref, avg_ref, o_ref):
    gw = g_ref[...]
    hw = h_ref[...]
    glo = lax.bitcast_convert_type(gw << 16, jnp.float32)
    ghi = lax.bitcast_convert_type(gw & _MASK_HI, jnp.float32)
    hlo = lax.bitcast_convert_type(hw << 16, jnp.float32)
    hhi = lax.bitcast_convert_type(hw & _MASK_HI, jnp.float32)

    efc_t = jnp.concatenate(
        [e0_ref[...], e1_ref[...], e2_ref[...], e3_ref[...]], axis=0)
    dn = (((0,), (0,)), ((), ()))
    zlo = lax.dot_general(efc_t, w1lo_ref[...], dn,
                          preferred_element_type=jnp.float32)
    zhi = lax.dot_general(efc_t, w1hi_ref[...], dn,
                          preferred_element_type=jnp.float32)
    zlo = zlo + glo + hlo + b1lo_ref[...]
    zhi = zhi + ghi + hhi + b1hi_ref[...]
    alo = zlo * jax.nn.sigmoid(zlo)
    ahi = zhi * jax.nn.sigmoid(zhi)
    o = (jnp.dot(alo, w2lo_ref[...], preferred_element_type=jnp.float32)
         + jnp.dot(ahi, w2hi_ref[...], preferred_element_type=jnp.float32)
         + b2_ref[...])
    avg = avg_ref[...]
    mu = jnp.dot(o, avg, preferred_element_type=jnp.float32)
    c = o - mu
    var = jnp.dot(c * c, avg, preferred_element_type=jnp.float32)
    y = c * lax.rsqrt(var + 1e-5) * gam_ref[...] + bet_ref[...]
    out_d = y.shape[-1] // 4
    for k in range(4):
        o_ref[k] = y[:, k * out_d:(k + 1) * out_d]


def _bd4(m):
    return jax.scipy.linalg.block_diag(m, m, m, m)


def _mlp(g4, h4, ef_t, w1e, b1, w2, b2, gamma, beta, blk):
    e4, wide = g4.shape
    efd, e = ef_t.shape
    hid = w1e.shape[1]
    hh = hid // 2
    out_d = w2.shape[1]
    grid = e4 // blk
    qblk = e // 4 // blk

    w1lo = _bd4(w1e[:, :hh])
    w1hi = _bd4(w1e[:, hh:])
    b1lo = jnp.tile(b1[:hh], 4).reshape(1, 4 * hh)
    b1hi = jnp.tile(b1[hh:], 4).reshape(1, 4 * hh)
    w2lo = _bd4(w2[:hh])
    w2hi = _bd4(w2[hh:])
    b2_4 = jnp.tile(b2, 4).reshape(1, 4 * out_d)
    gam4 = jnp.tile(gamma, 4).reshape(1, 4 * out_d)
    bet4 = jnp.tile(beta, 4).reshape(1, 4 * out_d)
    avg4 = _bd4(jnp.full((out_d, out_d), 1.0 / out_d, dtype=jnp.float32))

    ef_spec = [
        pl.BlockSpec((efd, blk), lambda i, k=k: (0, i + k * qblk))
        for k in range(4)
    ]
    y4 = pl.pallas_call(
        _mlp_body,
        grid=(grid,),
        in_specs=[
            pl.BlockSpec((blk, wide), lambda i: (i, 0)),
            pl.BlockSpec((blk, wide), lambda i: (i, 0)),
            *ef_spec,
            pl.BlockSpec((4 * efd, 4 * hh), lambda i: (0, 0)),
            pl.BlockSpec((4 * efd, 4 * hh), lambda i: (0, 0)),
            pl.BlockSpec((1, 4 * hh), lambda i: (0, 0)),
            pl.BlockSpec((1, 4 * hh), lambda i: (0, 0)),
            pl.BlockSpec((4 * hh, 4 * out_d), lambda i: (0, 0)),
            pl.BlockSpec((4 * hh, 4 * out_d), lambda i: (0, 0)),
            pl.BlockSpec((1, 4 * out_d), lambda i: (0, 0)),
            pl.BlockSpec((1, 4 * out_d), lambda i: (0, 0)),
            pl.BlockSpec((1, 4 * out_d), lambda i: (0, 0)),
            pl.BlockSpec((4 * out_d, 4 * out_d), lambda i: (0, 0)),
        ],
        out_specs=pl.BlockSpec((4, blk, out_d), lambda i: (0, i, 0)),
        out_shape=jax.ShapeDtypeStruct((4, e4, out_d), jnp.float32),
    )(g4, h4, ef_t, ef_t, ef_t, ef_t, w1lo, w1hi, b1lo, b1hi,
      w2lo, w2hi, b2_4, gam4, bet4, avg4)
    return y4.reshape(e, out_d)


def kernel(efeat, nfeat, edge_index, W1, b1, W2, b2, gamma, beta):
    e, efd = efeat.shape
    n, nfd = nfeat.shape
    hid = W1.shape[1]
    hh = hid // 2

    w1e = W1[:efd]
    w1s = W1[efd:efd + nfd]
    w1d = W1[efd + nfd:]

    ps_p, pd_p = _project_packed(nfeat, w1s, w1d)

    e_per_w = e // NW
    n_chunks = e_per_w // CHUNK
    src_q = edge_index[0].astype(jnp.int32).reshape(4, NW, e_per_w // 4)
    dst_q = edge_index[1].astype(jnp.int32).reshape(4, NW, e_per_w // 4)

    g, h = _make_gather(n, hid // 2, e, n_chunks)(ps_p, pd_p, src_q, dst_q)
    g4 = g.reshape(e // 4, 2 * hid)
    h4 = h.reshape(e // 4, 2 * hid)

    return _mlp(g4, h4, efeat.T, w1e, b1, W2, b2, gamma, beta, blk=3200)


# speedup vs baseline: 9.2092x; 1.3073x over previous
import functools

import jax
import jax.numpy as jnp
from jax import lax
from jax.experimental import pallas as pl
from jax.experimental.pallas import tpu as pltpu
from jax.experimental.pallas import tpu_sc as plsc

NW = 32
CHUNK = 80
K_CH = 5
GROUP = K_CH * CHUNK
LANES = 16


def _rn_bf16_hi(x):
    u = lax.bitcast_convert_type(x, jnp.int32)
    r = u + 0x7FFF + ((u >> 16) & 1)
    return r & _MASK_HI


def _proj_body(nf_ref, wsl_ref, wsh_ref, wdl_ref, wdh_ref, ps_ref, pd_ref):
    nf = nf_ref[...]

    def pack(wl_ref, wh_ref):
        zl = jnp.dot(nf, wl_ref[...], preferred_element_type=jnp.float32)
        zh = jnp.dot(nf, wh_ref[...], preferred_element_type=jnp.float32)
        lo = lax.shift_right_logical(_rn_bf16_hi(zl), 16)
        return _rn_bf16_hi(zh) | lo

    ps_ref[...] = pack(wsl_ref, wsh_ref)
    pd_ref[...] = pack(wdl_ref, wdh_ref)


def _project_packed(nfeat, w1s, w1d):
    n, _ = nfeat.shape
    hh = w1s.shape[1] // 2
    out = jax.ShapeDtypeStruct((n, hh), jnp.int32)
    return pl.pallas_call(_proj_body, out_shape=(out, out))(
        nfeat, w1s[:, :hh], w1s[:, hh:], w1d[:, :hh], w1d[:, hh:])


def _make_gather(n, hw, e, n_chunks):
    e_per_w = e // NW
    q_per_w = e_per_w // 4
    n_groups = n_chunks // K_CH
    n_col_v = CHUNK // LANES
    mesh = plsc.VectorSubcoreMesh(core_axis_name="c", subcore_axis_name="s")

    @functools.partial(
        pl.kernel,
        mesh=mesh,
        compiler_params=pltpu.CompilerParams(
            use_tc_tiling_on_sc=False, needs_layout_passes=False),
        out_type=(
            jax.ShapeDtypeStruct((e, hw), jnp.int32),
            jax.ShapeDtypeStruct((e, hw), jnp.int32),
        ),
        scratch_types=[
            pltpu.VMEM((n_chunks, CHUNK), jnp.int32),
            pltpu.VMEM((n_chunks, CHUNK), jnp.int32),
            pltpu.VMEM((4, q_per_w), jnp.int32),
            pltpu.VMEM((GROUP, hw), jnp.int32),
            pltpu.VMEM((GROUP, hw), jnp.int32),
            pltpu.VMEM((GROUP, hw), jnp.int32),
            pltpu.VMEM((GROUP, hw), jnp.int32),
            pltpu.SemaphoreType.DMA,
            pltpu.SemaphoreType.DMA,
            pltpu.SemaphoreType.DMA,
            pltpu.SemaphoreType.DMA,
        ],
    )
    def gather(ps_hbm, pd_hbm, src_hbm, dst_hbm, g_hbm, h_hbm,
               sidx, didx, qbuf, a0, b0, a1, b1, sa0, sb0, sa1, sb1):
        wid = lax.axis_index("s") * 2 + lax.axis_index("c")
        base = wid * e_per_w
        lanes = lax.iota(jnp.int32, LANES)

        kv = lanes & 3
        qv = lanes >> 2
        qp4 = CHUNK // 4

        def interleave(q_hbm, idx):
            for k in range(4):
                pltpu.sync_copy(q_hbm.at[k].at[wid], qbuf.at[k])

            def row(c, carry):
                for v in range(n_col_v):
                    qidx = c * qp4 + (LANES // 4) * v + qv
                    x = plsc.load_gather(qbuf, [kv, qidx])
                    idx[c, pl.ds(LANES * v, LANES)] = x
                return carry

            lax.fori_loop(0, n_chunks, row, 0)

        interleave(src_hbm, sidx)
        interleave(dst_hbm, didx)

        def fire(grp, abuf, bbuf, sa, sb):
            for k in range(K_CH):
                c = grp * K_CH + k
                sl = pl.ds(k * CHUNK, CHUNK)
                pltpu.async_copy(ps_hbm.at[sidx.at[c]], abuf.at[sl], sa)
                pltpu.async_copy(pd_hbm.at[didx.at[c]], bbuf.at[sl], sb)

        def drain_write(grp, abuf, bbuf, sa, sb):
            pltpu.make_async_copy(g_hbm.at[pl.ds(0, GROUP)], abuf, sa).wait()
            pltpu.make_async_copy(h_hbm.at[pl.ds(0, GROUP)], bbuf, sb).wait()
            row = base + grp * GROUP
            pltpu.sync_copy(abuf, g_hbm.at[pl.ds(row, GROUP)])
            pltpu.sync_copy(bbuf, h_hbm.at[pl.ds(row, GROUP)])

        fire(0, a0, b0, sa0, sb0)

        def body(i, carry):
            g0 = 2 * i
            g1 = g0 + 1
            g2 = g0 + 2

            @pl.when(g1 < n_groups)
            def _():
                fire(g1, a1, b1, sa1, sb1)

            drain_write(g0, a0, b0, sa0, sb0)

            @pl.when(g2 < n_groups)
            def _():
                fire(g2, a0, b0, sa0, sb0)

            @pl.when(g1 < n_groups)
            def _():
                drain_write(g1, a1, b1, sa1, sb1)

            return carry

        lax.fori_loop(0, (n_groups + 1) // 2, body, 0)

    return gather


_MASK_HI = -65536


def _mlp_body(g_ref, h_ref, e0_ref, e1_ref, e2_ref, e3_ref,
              w1lo_ref, w1hi_ref, b1lo_ref, b1hi_ref,
              w2lo_ref, w2hi_ref, b2_ref, gam_ref, bet_ref, avg_ref, o_ref):
    gw = g_ref[...]
    hw = h_ref[...]
    glo = lax.bitcast_convert_type(gw << 16, jnp.float32)
    ghi = lax.bitcast_convert_type(gw & _MASK_HI, jnp.float32)
    hlo = lax.bitcast_convert_type(hw << 16, jnp.float32)
    hhi = lax.bitcast_convert_type(hw & _MASK_HI, jnp.float32)

    efc_t = jnp.concatenate(
        [e0_ref[...], e1_ref[...], e2_ref[...], e3_ref[...]], axis=0)
    dn = (((0,), (0,)), ((), ()))
    zlo = lax.dot_general(efc_t, w1lo_ref[...], dn,
                          preferred_element_type=jnp.float32)
    zhi = lax.dot_general(efc_t, w1hi_ref[...], dn,
                          preferred_element_type=jnp.float32)
    zlo = zlo + glo + hlo + b1lo_ref[...]
    zhi = zhi + ghi + hhi + b1hi_ref[...]
    alo = zlo * jax.nn.sigmoid(zlo)
    ahi = zhi * jax.nn.sigmoid(zhi)
    dn_t = (((0,), (1,)), ((), ()))
    ot = (lax.dot_general(w2lo_ref[...], alo, dn_t,
                          preferred_element_type=jnp.float32)
          + lax.dot_general(w2hi_ref[...], ahi, dn_t,
                            preferred_element_type=jnp.float32)
          + b2_ref[...])
    avg = avg_ref[...]
    mut = lax.dot_general(avg, ot, dn,
                          preferred_element_type=jnp.float32)
    ct = ot - mut
    vart = lax.dot_general(avg, ct * ct, dn,
                           preferred_element_type=jnp.float32)
    yt = ct * lax.rsqrt(vart + 1e-5) * gam_ref[...] + bet_ref[...]
    out_d = yt.shape[0] // 4
    for k in range(4):
        o_ref[k] = yt[k * out_d:(k + 1) * out_d, :]


def _bd4(m):
    return jax.scipy.linalg.block_diag(m, m, m, m)


def _mlp(g4, h4, ef_t, w1e, b1, w2, b2, gamma, beta, blk):
    e4, wide = g4.shape
    efd, e = ef_t.shape
    hid = w1e.shape[1]
    hh = hid // 2
    out_d = w2.shape[1]
    grid = e4 // blk
    qblk = e // 4 // blk

    w1lo = _bd4(w1e[:, :hh])
    w1hi = _bd4(w1e[:, hh:])
    b1lo = jnp.tile(b1[:hh], 4).reshape(1, 4 * hh)
    b1hi = jnp.tile(b1[hh:], 4).reshape(1, 4 * hh)
    w2lo = _bd4(w2[:hh])
    w2hi = _bd4(w2[hh:])
    b2_4 = jnp.tile(b2, 4).reshape(4 * out_d, 1)
    gam4 = jnp.tile(gamma, 4).reshape(4 * out_d, 1)
    bet4 = jnp.tile(beta, 4).reshape(4 * out_d, 1)
    avg4 = _bd4(jnp.full((out_d, out_d), 1.0 / out_d, dtype=jnp.float32))

    ef_spec = [
        pl.BlockSpec((efd, blk), lambda i, k=k: (0, i + k * qblk))
        for k in range(4)
    ]
    y4 = pl.pallas_call(
        _mlp_body,
        grid=(grid,),
        in_specs=[
            pl.BlockSpec((blk, wide), lambda i: (i, 0)),
            pl.BlockSpec((blk, wide), lambda i: (i, 0)),
            *ef_spec,
            pl.BlockSpec((4 * efd, 4 * hh), lambda i: (0, 0)),
            pl.BlockSpec((4 * efd, 4 * hh), lambda i: (0, 0)),
            pl.BlockSpec((1, 4 * hh), lambda i: (0, 0)),
            pl.BlockSpec((1, 4 * hh), lambda i: (0, 0)),
            pl.BlockSpec((4 * hh, 4 * out_d), lambda i: (0, 0)),
            pl.BlockSpec((4 * hh, 4 * out_d), lambda i: (0, 0)),
            pl.BlockSpec((4 * out_d, 1), lambda i: (0, 0)),
            pl.BlockSpec((4 * out_d, 1), lambda i: (0, 0)),
            pl.BlockSpec((4 * out_d, 1), lambda i: (0, 0)),
            pl.BlockSpec((4 * out_d, 4 * out_d), lambda i: (0, 0)),
        ],
        out_specs=pl.BlockSpec((4, out_d, blk), lambda i: (0, 0, i)),
        out_shape=jax.ShapeDtypeStruct((4, out_d, e4), jnp.float32),
    )(g4, h4, ef_t, ef_t, ef_t, ef_t, w1lo, w1hi, b1lo, b1hi,
      w2lo, w2hi, b2_4, gam4, bet4, avg4)
    return jnp.transpose(y4, (1, 0, 2)).reshape(out_d, e).T


def kernel(efeat, nfeat, edge_index, W1, b1, W2, b2, gamma, beta):
    e, efd = efeat.shape
    n, nfd = nfeat.shape
    hid = W1.shape[1]
    hh = hid // 2

    w1e = W1[:efd]
    w1s = W1[efd:efd + nfd]
    w1d = W1[efd + nfd:]

    ps_p, pd_p = _project_packed(nfeat, w1s, w1d)

    e_per_w = e // NW
    n_chunks = e_per_w // CHUNK
    src_q = edge_index[0].astype(jnp.int32).reshape(4, NW, e_per_w // 4)
    dst_q = edge_index[1].astype(jnp.int32).reshape(4, NW, e_per_w // 4)

    g, h = _make_gather(n, hid // 2, e, n_chunks)(ps_p, pd_p, src_q, dst_q)
    g4 = g.reshape(e // 4, 2 * hid)
    h4 = h.reshape(e // 4, 2 * hid)

    return _mlp(g4, h4, efeat.T, w1e, b1, W2, b2, gamma, beta, blk=3200)


# speedup vs baseline: 9.6989x; 1.0532x over previous
import functools

import jax
import jax.numpy as jnp
from jax import lax
from jax.experimental import pallas as pl
from jax.experimental.pallas import tpu as pltpu
from jax.experimental.pallas import tpu_sc as plsc

NW = 32
CHUNK = 80
K_CH = 5
GROUP = K_CH * CHUNK
LANES = 16


def _rn_bf16_hi(x):
    u = lax.bitcast_convert_type(x, jnp.int32)
    r = u + 0x7FFF + ((u >> 16) & 1)
    return r & _MASK_HI


def _proj_body(nf_ref, wsl_ref, wsh_ref, wdl_ref, wdh_ref, ps_ref, pd_ref):
    nf = nf_ref[...]

    def pack(wl_ref, wh_ref):
        zl = jnp.dot(nf, wl_ref[...], preferred_element_type=jnp.float32)
        zh = jnp.dot(nf, wh_ref[...], preferred_element_type=jnp.float32)
        lo = lax.shift_right_logical(_rn_bf16_hi(zl), 16)
        return _rn_bf16_hi(zh) | lo

    ps_ref[...] = pack(wsl_ref, wsh_ref)
    pd_ref[...] = pack(wdl_ref, wdh_ref)


def _project_packed(nfeat, w1s, w1d):
    n, _ = nfeat.shape
    hh = w1s.shape[1] // 2
    out = jax.ShapeDtypeStruct((n, hh), jnp.int32)
    return pl.pallas_call(_proj_body, out_shape=(out, out))(
        nfeat, w1s[:, :hh], w1s[:, hh:], w1d[:, :hh], w1d[:, hh:])


def _make_gather(n, hw, e, n_chunks):
    e_per_w = e // NW
    q_per_w = e_per_w // 4
    n_groups = n_chunks // K_CH
    n_col_v = CHUNK // LANES
    mesh = plsc.VectorSubcoreMesh(core_axis_name="c", subcore_axis_name="s")

    @functools.partial(
        pl.kernel,
        mesh=mesh,
        compiler_params=pltpu.CompilerParams(
            use_tc_tiling_on_sc=False, needs_layout_passes=False),
        out_type=(
            jax.ShapeDtypeStruct((e, hw), jnp.int32),
            jax.ShapeDtypeStruct((e, hw), jnp.int32),
        ),
        scratch_types=[
            pltpu.VMEM((n_chunks, CHUNK), jnp.int32),
            pltpu.VMEM((n_chunks, CHUNK), jnp.int32),
            pltpu.VMEM((4, q_per_w), jnp.int32),
            pltpu.VMEM((GROUP, hw), jnp.int32),
            pltpu.VMEM((GROUP, hw), jnp.int32),
            pltpu.VMEM((GROUP, hw), jnp.int32),
            pltpu.VMEM((GROUP, hw), jnp.int32),
            pltpu.SemaphoreType.DMA,
            pltpu.SemaphoreType.DMA,
            pltpu.SemaphoreType.DMA,
            pltpu.SemaphoreType.DMA,
        ],
    )
    def gather(ps_hbm, pd_hbm, ei_hbm, g_hbm, h_hbm,
               sidx, didx, qbuf, a0, b0, a1, b1, sa0, sb0, sa1, sb1):
        wid = lax.axis_index("s") * 2 + lax.axis_index("c")
        base = wid * e_per_w
        lanes = lax.iota(jnp.int32, LANES)

        kv = lanes & 3
        qv = lanes >> 2
        qp4 = CHUNK // 4

        def interleave(s, idx):
            for k in range(4):
                pltpu.sync_copy(ei_hbm.at[s].at[k].at[wid], qbuf.at[k])

            def row(c, carry):
                for v in range(n_col_v):
                    qidx = c * qp4 + (LANES // 4) * v + qv
                    x = plsc.load_gather(qbuf, [kv, qidx])
                    idx[c, pl.ds(LANES * v, LANES)] = x
                return carry

            lax.fori_loop(0, n_chunks, row, 0)

        interleave(0, sidx)
        interleave(1, didx)

        def fire(grp, abuf, bbuf, sa, sb):
            for k in range(K_CH):
                c = grp * K_CH + k
                sl = pl.ds(k * CHUNK, CHUNK)
                pltpu.async_copy(ps_hbm.at[sidx.at[c]], abuf.at[sl], sa)
                pltpu.async_copy(pd_hbm.at[didx.at[c]], bbuf.at[sl], sb)

        def drain_write(grp, abuf, bbuf, sa, sb):
            pltpu.make_async_copy(g_hbm.at[pl.ds(0, GROUP)], abuf, sa).wait()
            pltpu.make_async_copy(h_hbm.at[pl.ds(0, GROUP)], bbuf, sb).wait()
            row = base + grp * GROUP
            pltpu.sync_copy(abuf, g_hbm.at[pl.ds(row, GROUP)])
            pltpu.sync_copy(bbuf, h_hbm.at[pl.ds(row, GROUP)])

        fire(0, a0, b0, sa0, sb0)

        def body(i, carry):
            g0 = 2 * i
            g1 = g0 + 1
            g2 = g0 + 2

            @pl.when(g1 < n_groups)
            def _():
                fire(g1, a1, b1, sa1, sb1)

            drain_write(g0, a0, b0, sa0, sb0)

            @pl.when(g2 < n_groups)
            def _():
                fire(g2, a0, b0, sa0, sb0)

            @pl.when(g1 < n_groups)
            def _():
                drain_write(g1, a1, b1, sa1, sb1)

            return carry

        lax.fori_loop(0, (n_groups + 1) // 2, body, 0)

    return gather


_MASK_HI = -65536


def _mlp_body(g_ref, h_ref, e0_ref, e1_ref, e2_ref, e3_ref,
              w1lo_ref, w1hi_ref, b1lo_ref, b1hi_ref,
              w2lo_ref, w2hi_ref, b2_ref, gam_ref, bet_ref, avg_ref, o_ref):
    gw = g_ref[...]
    hw = h_ref[...]
    glo = lax.bitcast_convert_type(gw << 16, jnp.float32)
    ghi = lax.bitcast_convert_type(gw & _MASK_HI, jnp.float32)
    hlo = lax.bitcast_convert_type(hw << 16, jnp.float32)
    hhi = lax.bitcast_convert_type(hw & _MASK_HI, jnp.float32)

    efc_t = jnp.concatenate(
        [e0_ref[...], e1_ref[...], e2_ref[...], e3_ref[...]], axis=0)
    dn = (((0,), (0,)), ((), ()))
    zlo = lax.dot_general(efc_t, w1lo_ref[...], dn,
                          preferred_element_type=jnp.float32)
    zhi = lax.dot_general(efc_t, w1hi_ref[...], dn,
                          preferred_element_type=jnp.float32)
    zlo = zlo + glo + hlo + b1lo_ref[...]
    zhi = zhi + ghi + hhi + b1hi_ref[...]
    alo = zlo * jax.nn.sigmoid(zlo)
    ahi = zhi * jax.nn.sigmoid(zhi)
    dn_t = (((0,), (1,)), ((), ()))
    ot = (lax.dot_general(w2lo_ref[...], alo, dn_t,
                          preferred_element_type=jnp.float32)
          + lax.dot_general(w2hi_ref[...], ahi, dn_t,
                            preferred_element_type=jnp.float32)
          + b2_ref[...])
    avg = avg_ref[...]
    mut = lax.dot_general(avg, ot, dn,
                          preferred_element_type=jnp.float32)
    ct = ot - mut
    vart = lax.dot_general(avg, ct * ct, dn,
                           preferred_element_type=jnp.float32)
    yt = ct * lax.rsqrt(vart + 1e-5) * gam_ref[...] + bet_ref[...]
    out_d = yt.shape[0] // 4
    for k in range(4):
        o_ref[k] = yt[k * out_d:(k + 1) * out_d, :]


def _bd4(m):
    return jax.scipy.linalg.block_diag(m, m, m, m)


def _mlp(g4, h4, ef_t, w1e, b1, w2, b2, gamma, beta, blk):
    e4, wide = g4.shape
    efd, e = ef_t.shape
    hid = w1e.shape[1]
    hh = hid // 2
    out_d = w2.shape[1]
    grid = e4 // blk
    qblk = e // 4 // blk

    w1lo = _bd4(w1e[:, :hh])
    w1hi = _bd4(w1e[:, hh:])
    b1lo = jnp.tile(b1[:hh], 4).reshape(1, 4 * hh)
    b1hi = jnp.tile(b1[hh:], 4).reshape(1, 4 * hh)
    w2lo = _bd4(w2[:hh])
    w2hi = _bd4(w2[hh:])
    b2_4 = jnp.tile(b2, 4).reshape(4 * out_d, 1)
    gam4 = jnp.tile(gamma, 4).reshape(4 * out_d, 1)
    bet4 = jnp.tile(beta, 4).reshape(4 * out_d, 1)
    avg4 = _bd4(jnp.full((out_d, out_d), 1.0 / out_d, dtype=jnp.float32))

    ef_spec = [
        pl.BlockSpec((efd, blk), lambda i, k=k: (0, i + k * qblk))
        for k in range(4)
    ]
    y4 = pl.pallas_call(
        _mlp_body,
        grid=(grid,),
        in_specs=[
            pl.BlockSpec((blk, wide), lambda i: (i, 0)),
            pl.BlockSpec((blk, wide), lambda i: (i, 0)),
            *ef_spec,
            pl.BlockSpec((4 * efd, 4 * hh), lambda i: (0, 0)),
            pl.BlockSpec((4 * efd, 4 * hh), lambda i: (0, 0)),
            pl.BlockSpec((1, 4 * hh), lambda i: (0, 0)),
            pl.BlockSpec((1, 4 * hh), lambda i: (0, 0)),
            pl.BlockSpec((4 * hh, 4 * out_d), lambda i: (0, 0)),
            pl.BlockSpec((4 * hh, 4 * out_d), lambda i: (0, 0)),
            pl.BlockSpec((4 * out_d, 1), lambda i: (0, 0)),
            pl.BlockSpec((4 * out_d, 1), lambda i: (0, 0)),
            pl.BlockSpec((4 * out_d, 1), lambda i: (0, 0)),
            pl.BlockSpec((4 * out_d, 4 * out_d), lambda i: (0, 0)),
        ],
        out_specs=pl.BlockSpec((4, out_d, blk), lambda i: (0, 0, i)),
        out_shape=jax.ShapeDtypeStruct((4, out_d, e4), jnp.float32),
    )(g4, h4, ef_t, ef_t, ef_t, ef_t, w1lo, w1hi, b1lo, b1hi,
      w2lo, w2hi, b2_4, gam4, bet4, avg4)
    return jnp.transpose(y4, (1, 0, 2)).reshape(out_d, e).T


def kernel(efeat, nfeat, edge_index, W1, b1, W2, b2, gamma, beta):
    e, efd = efeat.shape
    n, nfd = nfeat.shape
    hid = W1.shape[1]
    hh = hid // 2

    w1e = W1[:efd]
    w1s = W1[efd:efd + nfd]
    w1d = W1[efd + nfd:]

    ps_p, pd_p = _project_packed(nfeat, w1s, w1d)

    e_per_w = e // NW
    n_chunks = e_per_w // CHUNK
    ei = edge_index.astype(jnp.int32).reshape(2, 4, NW, e_per_w // 4)

    g, h = _make_gather(n, hid // 2, e, n_chunks)(ps_p, pd_p, ei)
    g4 = g.reshape(e // 4, 2 * hid)
    h4 = h.reshape(e // 4, 2 * hid)

    return _mlp(g4, h4, efeat.T, w1e, b1, W2, b2, gamma, beta, blk=3200)
